# Initial kernel scaffold; baseline (speedup 1.0000x reference)
import jax
import jax.numpy as jnp
from jax.experimental import pallas as pl


def kernel(x, edge_index, W1, b1, W2, b2):
    raise NotImplementedError("write your pallas kernel here")


# speedup vs baseline: 12.9080x; 12.9080x over previous
import functools

import jax
import jax.numpy as jnp
from jax import lax
from jax.experimental import pallas as pl
from jax.experimental.pallas import tpu as pltpu
from jax.experimental.pallas import tpu_sc as plsc

N = 10000
D = 128
E = 320000
NC = 2
NS = 16
NW = NC * NS
EPT = E // NW
ECH = 80
NCH = EPT // ECH
ROWS_PER_TILE = 624
TAIL_ROWS = N - 15 * ROWS_PER_TILE

_mesh = functools.partial(
    plsc.VectorSubcoreMesh,
    core_axis_name="c",
    subcore_axis_name="s",
    num_cores=NC,
    num_subcores=NS,
)


def _degree_body(dst_hbm, zeros_hbm, out_hbm, idx_v, ones_v, deg_sh, gsem):
  c = lax.axis_index("c")
  s = lax.axis_index("s")
  w = c * NS + s

  def fill(i, carry):
    ones_v[i, :] = jnp.ones((16,), jnp.float32)
    return carry
  lax.fori_loop(0, ECH, fill, 0)

  r0 = s * ROWS_PER_TILE
  pltpu.sync_copy(zeros_hbm.at[pl.ds(r0, ROWS_PER_TILE)],
                  deg_sh.at[pl.ds(r0, ROWS_PER_TILE)])

  @pl.when(s == NS - 1)
  def _():
    pltpu.sync_copy(zeros_hbm.at[pl.ds(N - TAIL_ROWS, TAIL_ROWS)],
                    deg_sh.at[pl.ds(N - TAIL_ROWS, TAIL_ROWS)])

  plsc.subcore_barrier()

  base = w * EPT

  def body(k, carry):
    pltpu.sync_copy(dst_hbm.at[pl.ds(base + k * ECH, ECH)], idx_v)
    pltpu.sync_copy(ones_v, deg_sh.at[idx_v], add=True)
    return carry
  lax.fori_loop(0, NCH, body, 0)

  plsc.subcore_barrier()

  pltpu.sync_copy(deg_sh.at[pl.ds(r0, ROWS_PER_TILE)],
                  out_hbm.at[pl.ds(c * N + r0, ROWS_PER_TILE)])

  @pl.when(s == NS - 1)
  def _():
    pltpu.sync_copy(deg_sh.at[pl.ds(N - TAIL_ROWS, TAIL_ROWS)],
                    out_hbm.at[pl.ds(c * N + N - TAIL_ROWS, TAIL_ROWS)])


_degree_sc = functools.partial(
    pl.kernel,
    out_type=jax.ShapeDtypeStruct((NC * N, 16), jnp.float32),
    mesh=_mesh(),
    scratch_types=[
        pltpu.VMEM((ECH,), jnp.int32),
        pltpu.VMEM((ECH, 16), jnp.float32),
        pltpu.VMEM_SHARED((N, 16), jnp.float32),
        pltpu.SemaphoreType.DMA,
    ],
)(_degree_body)


def _agg_body(table_hbm, src_hbm, dst_hbm, zeros_hbm, out_hbm,
              src_v, dst_v, rows_v, acc_sh, gsem):
  c = lax.axis_index("c")
  s = lax.axis_index("s")
  w = c * NS + s

  r0 = s * ROWS_PER_TILE
  pltpu.sync_copy(zeros_hbm.at[pl.ds(r0, ROWS_PER_TILE)],
                  acc_sh.at[pl.ds(r0, ROWS_PER_TILE)])

  @pl.when(s == NS - 1)
  def _():
    pltpu.sync_copy(zeros_hbm.at[pl.ds(N - TAIL_ROWS, TAIL_ROWS)],
                    acc_sh.at[pl.ds(N - TAIL_ROWS, TAIL_ROWS)])

  plsc.subcore_barrier()

  base = w * EPT

  def body(k, carry):
    off = base + k * ECH
    pltpu.sync_copy(src_hbm.at[pl.ds(off, ECH)], src_v)
    pltpu.sync_copy(dst_hbm.at[pl.ds(off, ECH)], dst_v)
    pltpu.async_copy(table_hbm.at[src_v], rows_v, gsem).wait()
    pltpu.sync_copy(rows_v, acc_sh.at[dst_v], add=True)
    return carry
  lax.fori_loop(0, NCH, body, 0)

  plsc.subcore_barrier()

  pltpu.sync_copy(acc_sh.at[pl.ds(r0, ROWS_PER_TILE)],
                  out_hbm.at[pl.ds(c * N + r0, ROWS_PER_TILE)])

  @pl.when(s == NS - 1)
  def _():
    pltpu.sync_copy(acc_sh.at[pl.ds(N - TAIL_ROWS, TAIL_ROWS)],
                    out_hbm.at[pl.ds(c * N + N - TAIL_ROWS, TAIL_ROWS)])


_agg_sc = functools.partial(
    pl.kernel,
    out_type=jax.ShapeDtypeStruct((NC * N, D), jnp.float32),
    mesh=_mesh(),
    scratch_types=[
        pltpu.VMEM((ECH,), jnp.int32),
        pltpu.VMEM((ECH,), jnp.int32),
        pltpu.VMEM((ECH, D), jnp.float32),
        pltpu.VMEM_SHARED((N, D), jnp.float32),
        pltpu.SemaphoreType.DMA,
    ],
)(_agg_body)


def _tc1_body(d0_ref, d1_ref, x_ref, w_ref, dis_ref, lins_ref):
  deg = 1.0 + d0_ref[...] + d1_ref[...]
  dis = lax.rsqrt(deg)
  dis_ref[...] = dis
  lins_ref[...] = jnp.dot(x_ref[...], w_ref[...],
                          preferred_element_type=jnp.float32) * dis


_tc1 = pl.pallas_call(
    _tc1_body,
    out_shape=[
        jax.ShapeDtypeStruct((N, 1), jnp.float32),
        jax.ShapeDtypeStruct((N, D), jnp.float32),
    ],
)


def _tc2_body(a0_ref, a1_ref, lins_ref, dis_ref, b_ref, w_ref, out_ref):
  h = (a0_ref[...] + a1_ref[...] + lins_ref[...]) * dis_ref[...] + b_ref[...]
  h = jnp.maximum(h, 0.0)
  out_ref[...] = jnp.dot(h, w_ref[...],
                         preferred_element_type=jnp.float32) * dis_ref[...]


_tc2 = pl.pallas_call(
    _tc2_body,
    out_shape=jax.ShapeDtypeStruct((N, D), jnp.float32),
)


def _tc3_body(a0_ref, a1_ref, lins_ref, dis_ref, b_ref, out_ref):
  out_ref[...] = ((a0_ref[...] + a1_ref[...] + lins_ref[...]) * dis_ref[...]
                  + b_ref[...])


_tc3 = pl.pallas_call(
    _tc3_body,
    out_shape=jax.ShapeDtypeStruct((N, D), jnp.float32),
)


def kernel(x, edge_index, W1, b1, W2, b2):
  e32 = edge_index.astype(jnp.int32)
  src = e32[0]
  dst = e32[1]
  zeros16 = jnp.zeros((N, 16), jnp.float32)
  zerosND = jnp.zeros((N, D), jnp.float32)
  b1r = b1.reshape(1, D)
  b2r = b2.reshape(1, D)

  degp = _degree_sc(dst, zeros16)
  d0 = degp[0:N, 0:1]
  d1 = degp[N:2 * N, 0:1]

  dis, lins1 = _tc1(d0, d1, x, W1)
  acc1 = _agg_sc(lins1, src, dst, zerosND)
  lins2 = _tc2(acc1[0:N], acc1[N:2 * N], lins1, dis, b1r, W2)
  acc2 = _agg_sc(lins2, src, dst, zerosND)
  return _tc3(acc2[0:N], acc2[N:2 * N], lins2, dis, b2r)


# speedup vs baseline: 27.3533x; 2.1191x over previous
import functools

import jax
import jax.numpy as jnp
from jax import lax
from jax.experimental import pallas as pl
from jax.experimental.pallas import tpu as pltpu
from jax.experimental.pallas import tpu_sc as plsc

N = 10000
D = 128
E = 320000
NC = 2
NS = 16
NW = NC * NS
EPT = E // NW
ECH = 125
NCH = EPT // ECH
ROWS_PER_TILE = 624
TAIL_ROWS = N - 15 * ROWS_PER_TILE

_mesh = functools.partial(
    plsc.VectorSubcoreMesh,
    core_axis_name="c",
    subcore_axis_name="s",
    num_cores=NC,
    num_subcores=NS,
)


def _degree_body(dst_hbm, zeros_hbm, out_hbm, idx_b, ones_v, deg_sh, ssem):
  c = lax.axis_index("c")
  s = lax.axis_index("s")
  w = c * NS + s

  def fill(i, carry):
    ones_v[i, :] = jnp.ones((16,), jnp.float32)
    return carry
  lax.fori_loop(0, ECH, fill, 0)

  pltpu.sync_copy(dst_hbm.at[w], idx_b)

  r0 = s * ROWS_PER_TILE
  pltpu.sync_copy(zeros_hbm.at[pl.ds(r0, ROWS_PER_TILE)],
                  deg_sh.at[pl.ds(r0, ROWS_PER_TILE)])

  @pl.when(s == NS - 1)
  def _():
    pltpu.sync_copy(zeros_hbm.at[pl.ds(N - TAIL_ROWS, TAIL_ROWS)],
                    deg_sh.at[pl.ds(N - TAIL_ROWS, TAIL_ROWS)])

  plsc.subcore_barrier()

  def fire(k, carry):
    pltpu.async_copy(ones_v, deg_sh.at[idx_b.at[k]], ssem, add=True)
    return carry
  lax.fori_loop(0, NCH, fire, 0)

  def drain(k, carry):
    pltpu.make_async_copy(ones_v, deg_sh.at[idx_b.at[0]], ssem).wait()
    return carry
  lax.fori_loop(0, NCH, drain, 0)

  plsc.subcore_barrier()

  pltpu.sync_copy(deg_sh.at[pl.ds(r0, ROWS_PER_TILE)],
                  out_hbm.at[pl.ds(c * N + r0, ROWS_PER_TILE)])

  @pl.when(s == NS - 1)
  def _():
    pltpu.sync_copy(deg_sh.at[pl.ds(N - TAIL_ROWS, TAIL_ROWS)],
                    out_hbm.at[pl.ds(c * N + N - TAIL_ROWS, TAIL_ROWS)])


_degree_sc = functools.partial(
    pl.kernel,
    out_type=jax.ShapeDtypeStruct((NC * N, 16), jnp.float32),
    mesh=_mesh(),
    scratch_types=[
        pltpu.VMEM((NCH, ECH), jnp.int32),
        pltpu.VMEM((ECH, 16), jnp.float32),
        pltpu.VMEM_SHARED((N, 16), jnp.float32),
        pltpu.SemaphoreType.DMA,
    ],
)(_degree_body)


def _agg_body(table_hbm, eidx_hbm, zeros_hbm, out_hbm,
              slot0, slot1, rows0, rows1, acc_sh,
              isem0, isem1, gsem0, gsem1):
  c = lax.axis_index("c")
  s = lax.axis_index("s")
  w = c * NS + s
  slots = (slot0, slot1)
  rows = (rows0, rows1)
  isems = (isem0, isem1)
  gsems = (gsem0, gsem1)

  pltpu.async_copy(eidx_hbm.at[w, 0], slot0, isem0)
  pltpu.async_copy(eidx_hbm.at[w, 1], slot1, isem1)

  r0 = s * ROWS_PER_TILE
  pltpu.sync_copy(zeros_hbm.at[pl.ds(r0, ROWS_PER_TILE)],
                  acc_sh.at[pl.ds(r0, ROWS_PER_TILE)])

  @pl.when(s == NS - 1)
  def _():
    pltpu.sync_copy(zeros_hbm.at[pl.ds(N - TAIL_ROWS, TAIL_ROWS)],
                    acc_sh.at[pl.ds(N - TAIL_ROWS, TAIL_ROWS)])

  plsc.subcore_barrier()

  pltpu.make_async_copy(eidx_hbm.at[w, 0], slot0, isem0).wait()
  pltpu.async_copy(table_hbm.at[slot0.at[0]], rows0, gsem0)

  def body(i2, carry):
    for par in range(2):
      i = i2 * 2 + par
      p, q = par, 1 - par
      pltpu.make_async_copy(table_hbm.at[slots[p].at[0]], rows[p],
                            gsems[p]).wait()

      @pl.when(i + 1 < NCH)
      def _():
        pltpu.make_async_copy(eidx_hbm.at[w, 0], slots[q], isems[q]).wait()
        pltpu.async_copy(table_hbm.at[slots[q].at[0]], rows[q], gsems[q])

      pltpu.sync_copy(rows[p], acc_sh.at[slots[p].at[1]], add=True)

      @pl.when(i + 2 < NCH)
      def _():
        pltpu.async_copy(eidx_hbm.at[w, i + 2], slots[p], isems[p])
    return carry
  lax.fori_loop(0, NCH // 2, body, 0)

  plsc.subcore_barrier()

  pltpu.sync_copy(acc_sh.at[pl.ds(r0, ROWS_PER_TILE)],
                  out_hbm.at[pl.ds(c * N + r0, ROWS_PER_TILE)])

  @pl.when(s == NS - 1)
  def _():
    pltpu.sync_copy(acc_sh.at[pl.ds(N - TAIL_ROWS, TAIL_ROWS)],
                    out_hbm.at[pl.ds(c * N + N - TAIL_ROWS, TAIL_ROWS)])


_agg_sc = functools.partial(
    pl.kernel,
    out_type=jax.ShapeDtypeStruct((NC * N, D), jnp.float32),
    mesh=_mesh(),
    scratch_types=[
        pltpu.VMEM((2, ECH), jnp.int32),
        pltpu.VMEM((2, ECH), jnp.int32),
        pltpu.VMEM((ECH, D), jnp.float32),
        pltpu.VMEM((ECH, D), jnp.float32),
        pltpu.VMEM_SHARED((N, D), jnp.float32),
        pltpu.SemaphoreType.DMA,
        pltpu.SemaphoreType.DMA,
        pltpu.SemaphoreType.DMA,
        pltpu.SemaphoreType.DMA,
    ],
)(_agg_body)


def _tc1_body(d0_ref, d1_ref, x_ref, w_ref, dis_ref, lins_ref):
  deg = 1.0 + d0_ref[...] + d1_ref[...]
  dis = lax.rsqrt(deg)
  dis_ref[...] = dis
  lins_ref[...] = jnp.dot(x_ref[...], w_ref[...],
                          preferred_element_type=jnp.float32) * dis


_tc1 = pl.pallas_call(
    _tc1_body,
    out_shape=[
        jax.ShapeDtypeStruct((N, 1), jnp.float32),
        jax.ShapeDtypeStruct((N, D), jnp.float32),
    ],
)


def _tc2_body(a0_ref, a1_ref, lins_ref, dis_ref, b_ref, w_ref, out_ref):
  h = (a0_ref[...] + a1_ref[...] + lins_ref[...]) * dis_ref[...] + b_ref[...]
  h = jnp.maximum(h, 0.0)
  out_ref[...] = jnp.dot(h, w_ref[...],
                         preferred_element_type=jnp.float32) * dis_ref[...]


_tc2 = pl.pallas_call(
    _tc2_body,
    out_shape=jax.ShapeDtypeStruct((N, D), jnp.float32),
)


def _tc3_body(a0_ref, a1_ref, lins_ref, dis_ref, b_ref, out_ref):
  out_ref[...] = ((a0_ref[...] + a1_ref[...] + lins_ref[...]) * dis_ref[...]
                  + b_ref[...])


_tc3 = pl.pallas_call(
    _tc3_body,
    out_shape=jax.ShapeDtypeStruct((N, D), jnp.float32),
)


def kernel(x, edge_index, W1, b1, W2, b2):
  e32 = edge_index.astype(jnp.int32)
  src = e32[0].reshape(NW, NCH, ECH)
  dst = e32[1].reshape(NW, NCH, ECH)
  eidx = jnp.stack([src, dst], axis=2)
  zeros16 = jnp.zeros((N, 16), jnp.float32)
  zerosND = jnp.zeros((N, D), jnp.float32)
  b1r = b1.reshape(1, D)
  b2r = b2.reshape(1, D)

  degp = _degree_sc(dst, zeros16)
  d0 = degp[0:N, 0:1]
  d1 = degp[N:2 * N, 0:1]

  dis, lins1 = _tc1(d0, d1, x, W1)
  acc1 = _agg_sc(lins1, eidx, zerosND)
  lins2 = _tc2(acc1[0:N], acc1[N:2 * N], lins1, dis, b1r, W2)
  acc2 = _agg_sc(lins2, eidx, zerosND)
  return _tc3(acc2[0:N], acc2[N:2 * N], lins2, dis, b2r)


# speedup vs baseline: 27.4244x; 1.0026x over previous
import functools

import jax
import jax.numpy as jnp
from jax import lax
from jax.experimental import pallas as pl
from jax.experimental.pallas import tpu as pltpu
from jax.experimental.pallas import tpu_sc as plsc

N = 10000
D = 128
E = 320000
NC = 2
NS = 16
NW = NC * NS
EPT = E // NW
ECH = 125
NCH = EPT // ECH
ROWS_PER_TILE = 624
TAIL_ROWS = N - 15 * ROWS_PER_TILE

_mesh = functools.partial(
    plsc.VectorSubcoreMesh,
    core_axis_name="c",
    subcore_axis_name="s",
    num_cores=NC,
    num_subcores=NS,
)


def _degree_body(dst_hbm, zeros_hbm, out_hbm, idx_b, ones_v, deg_sh, ssem):
  c = lax.axis_index("c")
  s = lax.axis_index("s")
  w = c * NS + s

  def fill(i, carry):
    ones_v[i, :] = jnp.ones((16,), jnp.float32)
    return carry
  lax.fori_loop(0, ECH, fill, 0)

  pltpu.sync_copy(dst_hbm.at[w], idx_b)

  r0 = s * ROWS_PER_TILE
  pltpu.sync_copy(zeros_hbm.at[pl.ds(r0, ROWS_PER_TILE)],
                  deg_sh.at[pl.ds(r0, ROWS_PER_TILE)])

  @pl.when(s == NS - 1)
  def _():
    pltpu.sync_copy(zeros_hbm.at[pl.ds(N - TAIL_ROWS, TAIL_ROWS)],
                    deg_sh.at[pl.ds(N - TAIL_ROWS, TAIL_ROWS)])

  plsc.subcore_barrier()

  def fire(k, carry):
    pltpu.async_copy(ones_v, deg_sh.at[idx_b.at[k]], ssem, add=True)
    return carry
  lax.fori_loop(0, NCH, fire, 0)

  def drain(k, carry):
    pltpu.make_async_copy(ones_v, deg_sh.at[idx_b.at[0]], ssem).wait()
    return carry
  lax.fori_loop(0, NCH, drain, 0)

  plsc.subcore_barrier()

  pltpu.sync_copy(deg_sh.at[pl.ds(r0, ROWS_PER_TILE)],
                  out_hbm.at[pl.ds(c * N + r0, ROWS_PER_TILE)])

  @pl.when(s == NS - 1)
  def _():
    pltpu.sync_copy(deg_sh.at[pl.ds(N - TAIL_ROWS, TAIL_ROWS)],
                    out_hbm.at[pl.ds(c * N + N - TAIL_ROWS, TAIL_ROWS)])


_degree_sc = functools.partial(
    pl.kernel,
    out_type=jax.ShapeDtypeStruct((NC * N, 16), jnp.float32),
    mesh=_mesh(),
    scratch_types=[
        pltpu.VMEM((NCH, ECH), jnp.int32),
        pltpu.VMEM((ECH, 16), jnp.float32),
        pltpu.VMEM_SHARED((N, 16), jnp.float32),
        pltpu.SemaphoreType.DMA,
    ],
)(_degree_body)


def _agg_body(table_hbm, eidx_hbm, zeros_hbm, out_hbm,
              slot0, slot1, slot2, rows0, rows1, rows2, acc_sh,
              isem0, isem1, isem2, gsem0, gsem1, gsem2,
              ssem0, ssem1, ssem2):
  c = lax.axis_index("c")
  s = lax.axis_index("s")
  w = c * NS + s
  slots = (slot0, slot1, slot2)
  rows = (rows0, rows1, rows2)
  isems = (isem0, isem1, isem2)
  gsems = (gsem0, gsem1, gsem2)
  ssems = (ssem0, ssem1, ssem2)

  def start_idx(i, par):
    pltpu.async_copy(eidx_hbm.at[w, i], slots[par], isems[par])

  def wait_idx(par):
    pltpu.make_async_copy(eidx_hbm.at[w, 0], slots[par], isems[par]).wait()

  def start_gather(par):
    pltpu.async_copy(table_hbm.at[slots[par].at[0]], rows[par], gsems[par])

  def wait_gather(par):
    pltpu.make_async_copy(table_hbm.at[slots[par].at[0]], rows[par],
                          gsems[par]).wait()

  def start_scatter(par):
    pltpu.async_copy(rows[par], acc_sh.at[slots[par].at[1]], ssems[par],
                     add=True)

  def wait_scatter(par):
    pltpu.make_async_copy(rows[par], acc_sh.at[slots[par].at[1]],
                          ssems[par]).wait()

  start_idx(0, 0)
  start_idx(1, 1)

  r0 = s * ROWS_PER_TILE
  pltpu.sync_copy(zeros_hbm.at[pl.ds(r0, ROWS_PER_TILE)],
                  acc_sh.at[pl.ds(r0, ROWS_PER_TILE)])

  @pl.when(s == NS - 1)
  def _():
    pltpu.sync_copy(zeros_hbm.at[pl.ds(N - TAIL_ROWS, TAIL_ROWS)],
                    acc_sh.at[pl.ds(N - TAIL_ROWS, TAIL_ROWS)])

  plsc.subcore_barrier()

  wait_idx(0)
  start_gather(0)

  def emit(i, par, in_loop):
    p1 = (par + 1) % 3
    p2 = (par + 2) % 3
    wait_gather(par)
    if in_loop:
      @pl.when(i >= 1)
      def _():
        wait_scatter(p2)
      start_scatter(par)
      start_idx(i + 2, p2)
      wait_idx(p1)
      start_gather(p1)
    else:
      if i >= 1:
        wait_scatter(p2)
      start_scatter(par)
      if i + 2 < NCH:
        start_idx(i + 2, p2)
      if i + 1 < NCH:
        wait_idx(p1)
        start_gather(p1)

  def body(i3, carry):
    for par in range(3):
      emit(i3 * 3 + par, par, True)
    return carry
  lax.fori_loop(0, (NCH - 2) // 3, body, 0)
  for i in range(NCH - 2, NCH):
    emit(i, i % 3, False)
  wait_scatter((NCH - 1) % 3)

  plsc.subcore_barrier()

  pltpu.sync_copy(acc_sh.at[pl.ds(r0, ROWS_PER_TILE)],
                  out_hbm.at[pl.ds(c * N + r0, ROWS_PER_TILE)])

  @pl.when(s == NS - 1)
  def _():
    pltpu.sync_copy(acc_sh.at[pl.ds(N - TAIL_ROWS, TAIL_ROWS)],
                    out_hbm.at[pl.ds(c * N + N - TAIL_ROWS, TAIL_ROWS)])


_agg_sc = functools.partial(
    pl.kernel,
    out_type=jax.ShapeDtypeStruct((NC * N, D), jnp.float32),
    mesh=_mesh(),
    scratch_types=(
        [pltpu.VMEM((2, ECH), jnp.int32)] * 3
        + [pltpu.VMEM((ECH, D), jnp.float32)] * 3
        + [pltpu.VMEM_SHARED((N, D), jnp.float32)]
        + [pltpu.SemaphoreType.DMA] * 9
    ),
)(_agg_body)


def _tc1_body(d0_ref, d1_ref, x_ref, w_ref, dis_ref, lins_ref):
  deg = 1.0 + d0_ref[...] + d1_ref[...]
  dis = lax.rsqrt(deg)
  dis_ref[...] = dis
  lins_ref[...] = jnp.dot(x_ref[...], w_ref[...],
                          preferred_element_type=jnp.float32) * dis


_tc1 = pl.pallas_call(
    _tc1_body,
    out_shape=[
        jax.ShapeDtypeStruct((N, 1), jnp.float32),
        jax.ShapeDtypeStruct((N, D), jnp.float32),
    ],
)


def _tc2_body(a0_ref, a1_ref, lins_ref, dis_ref, b_ref, w_ref, out_ref):
  h = (a0_ref[...] + a1_ref[...] + lins_ref[...]) * dis_ref[...] + b_ref[...]
  h = jnp.maximum(h, 0.0)
  out_ref[...] = jnp.dot(h, w_ref[...],
                         preferred_element_type=jnp.float32) * dis_ref[...]


_tc2 = pl.pallas_call(
    _tc2_body,
    out_shape=jax.ShapeDtypeStruct((N, D), jnp.float32),
)


def _tc3_body(a0_ref, a1_ref, lins_ref, dis_ref, b_ref, out_ref):
  out_ref[...] = ((a0_ref[...] + a1_ref[...] + lins_ref[...]) * dis_ref[...]
                  + b_ref[...])


_tc3 = pl.pallas_call(
    _tc3_body,
    out_shape=jax.ShapeDtypeStruct((N, D), jnp.float32),
)


def kernel(x, edge_index, W1, b1, W2, b2):
  e32 = edge_index.astype(jnp.int32)
  src = e32[0].reshape(NW, NCH, ECH)
  dst = e32[1].reshape(NW, NCH, ECH)
  eidx = jnp.stack([src, dst], axis=2)
  zeros16 = jnp.zeros((N, 16), jnp.float32)
  zerosND = jnp.zeros((N, D), jnp.float32)
  b1r = b1.reshape(1, D)
  b2r = b2.reshape(1, D)

  degp = _degree_sc(dst, zeros16)
  d0 = degp[0:N, 0:1]
  d1 = degp[N:2 * N, 0:1]

  dis, lins1 = _tc1(d0, d1, x, W1)
  acc1 = _agg_sc(lins1, eidx, zerosND)
  lins2 = _tc2(acc1[0:N], acc1[N:2 * N], lins1, dis, b1r, W2)
  acc2 = _agg_sc(lins2, eidx, zerosND)
  return _tc3(acc2[0:N], acc2[N:2 * N], lins2, dis, b2r)
